# baseline (speedup 1.0000x reference)
import functools

import jax
import jax.numpy as jnp
from jax.experimental import pallas as pl
from jax.experimental.pallas import tpu as pltpu
from jax.experimental.pallas import tpu_sc as plsc

_X = 512
_Y = 512
_Z = 32
_N = 1000
_NP = 1024
_THR = 0.5
_G = 32
_R = (_X * _Y * _Z) // (128 * _G)
_P = (_X * _Y) // _G


def _prep_kernel(pm_ref, sc_r_ref, b0_r_ref, b1_r_ref, b2_r_ref, b3_r_ref,
                 zl_r_ref, sc_c_ref, b0_c_ref, b1_c_ref, zl_c_ref,
                 pmbz_ref, box_ref, bz_ref, pmmax_ref, nv_ref,
                 bz_s, pmmax_s):
    s = pl.program_id(0)

    @pl.when(s == 0)
    def _():
        sc_r = sc_r_ref[...]
        zl_r = zl_r_ref[...]
        sc_c = sc_c_ref[...]
        zl_c = zl_c_ref[...]

        keep_r = sc_r > _THR
        keep_c = sc_c > _THR
        kf_r = keep_r.astype(jnp.float32)
        kf_c = keep_c.astype(jnp.float32)
        K = jnp.sum(kf_r).astype(jnp.int32)

        best = jnp.float32(-1.0)
        bz = jnp.int32(0)
        for zz in range(_Z):
            m = keep_r & (zl_r == zz)
            cnt = jnp.sum(m.astype(jnp.float32))
            tot = jnp.sum(jnp.where(m, sc_r, 0.0))
            mean = jnp.where(cnt > 0, tot / jnp.maximum(cnt, 1.0),
                             jnp.float32(0.0))
            upd = mean > best
            bz = jnp.where(upd, jnp.int32(zz), bz)
            best = jnp.maximum(mean, best)
        bz_s[0] = bz
        bz_ref[0, 0] = bz

        i0i = jax.lax.broadcasted_iota(jnp.int32, (_NP, _NP), 0)
        i1i = jax.lax.broadcasted_iota(jnp.int32, (_NP, _NP), 1)
        i0 = i0i.astype(jnp.float32)
        i1 = i1i.astype(jnp.float32)

        le = (i0 <= i1).astype(jnp.float32)
        ge = (i0 >= i1).astype(jnp.float32)
        rank_r = jax.lax.dot_general(kf_r, le, (((1,), (0,)), ((), ())),
                                     preferred_element_type=jnp.float32)
        rank_c = jax.lax.dot_general(ge, kf_c, (((1,), (0,)), ((), ())),
                                     preferred_element_type=jnp.float32)

        sel_a = (keep_r & (rank_r - 1.0 == i0)).astype(jnp.float32)
        xk_c = jnp.sum(sel_a * b0_r_ref[...], axis=1, keepdims=True)
        ki_c = jnp.sum(sel_a * i1, axis=1, keepdims=True)
        sel_b = (keep_c & (rank_c - 1.0 == i1)).astype(jnp.float32)
        xk_r = jnp.sum(sel_b * b0_c_ref[...], axis=0, keepdims=True)
        ki_r = jnp.sum(sel_b * i0, axis=0, keepdims=True)

        jr = jax.lax.broadcasted_iota(jnp.int32, (1, _NP), 1)
        jc = jax.lax.broadcasted_iota(jnp.int32, (_NP, 1), 0)

        def passed(jv, b0, b1, sc, zl):
            return ((jv < K) & (b1 <= _X) & (b0 <= _Y) & (sc >= _THR)
                    & (zl >= bz - 2) & (zl <= bz + 2))

        p_r = passed(jr, b0_r_ref[...], b1_r_ref[...], sc_r, zl_r)
        p_c = passed(jc, b0_c_ref[...], b1_c_ref[...], sc_c, zl_c)

        lt = ((xk_c < xk_r) | ((xk_c == xk_r) & (ki_c < ki_r))
              ).astype(jnp.float32)
        cid = 2.0 + jax.lax.dot_general(
            p_c.astype(jnp.float32), lt, (((0,), (0,)), ((), ())),
            preferred_element_type=jnp.float32)
        cid = cid.astype(jnp.int32)

        y1 = b0_r_ref[...]
        x1 = b1_r_ref[...]
        y2 = b2_r_ref[...]
        x2 = b3_r_ref[...]
        fX = jnp.float32(_X)
        fY = jnp.float32(_Y)
        x2c = jnp.where(x2 > fX, fX - 1.0, x2)
        y2c = jnp.where((x2 <= fX) & (y2 > fY), fY - 1.0, y2)
        dx0 = jnp.where(x1 + 5.0 < 0.0, x1, jnp.float32(5.0))
        dy0 = jnp.where(y1 + 5.0 < 0.0, y1, jnp.float32(5.0))
        dx1 = jnp.where(x2c - 5.0 > fX, fX - x2c, jnp.float32(-5.0))
        dy1 = jnp.where(y2c - 5.0 > fY, fY - y2c, jnp.float32(-5.0))
        xa = jnp.round(x1 + dx0).astype(jnp.int32)
        xb = jnp.round(x2c + dx1).astype(jnp.int32)
        ya = jnp.round(y1 + dy0).astype(jnp.int32)
        yb = jnp.round(y2c + dy1).astype(jnp.int32)
        xa = jnp.maximum(xa, 0)
        ya = jnp.maximum(ya, 0)
        xb = jnp.minimum(xb, _X)
        yb = jnp.minimum(yb, _Y)
        valid = p_r & (xb > xa) & (yb > ya)
        vf = valid.astype(jnp.float32)
        nv_ref[0, 0] = jnp.sum(vf).astype(jnp.int32)

        vrank = jax.lax.dot_general(vf, le, (((1,), (0,)), ((), ())),
                                    preferred_element_type=jnp.float32)
        csel = (valid & (vrank - 1.0 == i0)).astype(jnp.float32)

        def compact(arr_row):
            return jnp.sum(csel * arr_row.astype(jnp.float32), axis=1,
                           keepdims=True).astype(jnp.int32)

        box_ref[:, 0:1] = compact(xa)
        box_ref[:, 1:2] = compact(xb)
        box_ref[:, 2:3] = compact(ya)
        box_ref[:, 3:4] = compact(yb)
        box_ref[:, 4:5] = compact(cid)
        box_ref[:, 5:16] = jnp.zeros((_NP, 11), jnp.int32)
        box_ref[0:1, 5:6] = nv_ref[0, 0].reshape(1, 1)

    blk = pm_ref[0]
    blk_max = jnp.max(blk)

    @pl.when(s == 0)
    def _():
        pmmax_s[0] = blk_max

    @pl.when(s > 0)
    def _():
        pmmax_s[0] = jnp.maximum(pmmax_s[0], blk_max)

    @pl.when(s == _G - 1)
    def _():
        pmmax_ref[0, 0] = pmmax_s[0]

    li = jax.lax.broadcasted_iota(jnp.int32, (_R, 128), 1)
    bz = bz_s[0]
    parts = [
        jnp.sum(jnp.where(li == 32 * k + bz, blk, 0.0), axis=1,
                keepdims=True)
        for k in range(4)
    ]
    pmbz_ref[0] = jnp.concatenate(parts, axis=1)


def _boxes_kernel(pm_ref, box_ref, nv_ref, s_out_ref):
    s_out_ref[...] = jnp.zeros((_X, _Y), jnp.int32)

    def body(i, carry):
        xa = box_ref[i, 0]
        xb = box_ref[i, 1]
        ya = box_ref[i, 2]
        yb = box_ref[i, 3]
        cid = box_ref[i, 4]
        bx = jnp.minimum((xa // 8) * 8, _X - 64)
        by = jnp.minimum((ya // 128) * 128, _Y - 256)
        win = pm_ref[pl.ds(bx, 64), pl.ds(by, 256)]
        scur = s_out_ref[pl.ds(bx, 64), pl.ds(by, 256)]
        ix = jax.lax.broadcasted_iota(jnp.int32, (64, 256), 0) + bx
        iy = jax.lax.broadcasted_iota(jnp.int32, (64, 256), 1) + by
        rect = (ix >= xa) & (ix < xb) & (iy >= ya) & (iy < yb)
        m = jnp.max(jnp.where(rect, win, -jnp.inf))
        eq = rect & (win == m)
        s_out_ref[pl.ds(bx, 64), pl.ds(by, 256)] = jnp.where(
            eq, jnp.maximum(scur, cid), scur)
        return carry

    jax.lax.fori_loop(0, nv_ref[0, 0], body, 0)


def _sc_boxes_kernel(pm_hbm, box_hbm, out_hbm, boxv, winv, sref):
    w = jax.lax.axis_index("s") * 2 + jax.lax.axis_index("c")
    x0 = w * 16
    pltpu.sync_copy(box_hbm, boxv)
    nv = boxv[0, pl.ds(0, 16)][5]
    il = jax.lax.iota(jnp.int32, 16)

    def zb(i, c):
        sref[i // 32, pl.ds((i % 32) * 16, 16)] = jnp.zeros((16,), jnp.int32)
        return c

    jax.lax.fori_loop(0, 512, zb, 0)

    def body(b, c):
        row = boxv[b // 8, pl.ds((b % 8) * 16, 16)]
        xa = row[0]
        xb = row[1]
        ya = row[2]
        yb = row[3]
        cid = row[4]
        xs = jnp.maximum(xa, x0)
        xe = jnp.minimum(xb, x0 + 16)

        @pl.when(xs < xe)
        def _():
            bx = jnp.minimum((xa // 8) * 8, _X - 64)
            by = jnp.minimum((ya // 128) * 128, _Y - 256)
            c0 = (ya - by) // 16
            pltpu.sync_copy(pm_hbm.at[pl.ds(bx, 64), pl.ds(by, 256)], winv)

            def rmax(r, acc):
                def cmax(ci, acc2):
                    y0 = by + (c0 + ci) * 16
                    v = winv[r, pl.ds((c0 + ci) * 16, 16)]
                    ym = (il + y0 >= ya) & (il + y0 < yb)
                    return jnp.maximum(acc2, jnp.where(ym, v, -jnp.inf))

                return jax.lax.fori_loop(0, 5, cmax, acc)

            acc = jax.lax.fori_loop(xa - bx, xb - bx, rmax,
                                    jnp.full((16,), -jnp.inf, jnp.float32))
            dnums = jax.lax.GatherDimensionNumbers(
                offset_dims=(), collapsed_slice_dims=(0,),
                start_index_map=(0,))
            for sh in (8, 4, 2, 1):
                peer = jax.lax.gather(
                    acc, (il ^ sh)[:, None], dnums, (1,),
                    mode=jax.lax.GatherScatterMode.PROMISE_IN_BOUNDS)
                acc = jnp.maximum(acc, peer)
            m = acc

            def rupd(x, c2):
                def cupd(ci, c3):
                    y0 = by + (c0 + ci) * 16
                    v = winv[x - bx, pl.ds((c0 + ci) * 16, 16)]
                    ym = (il + y0 >= ya) & (il + y0 < yb)
                    eq = ym & (v == m)
                    scur = sref[x - x0, pl.ds(y0, 16)]
                    sref[x - x0, pl.ds(y0, 16)] = jnp.where(
                        eq, jnp.maximum(scur, cid), scur)
                    return c3

                return jax.lax.fori_loop(0, 5, cupd, 0)

            jax.lax.fori_loop(xs, xe, rupd, 0)

        return c

    jax.lax.fori_loop(0, nv, body, 0)
    pltpu.sync_copy(sref, out_hbm.at[pl.ds(x0, 16)])


def _assemble_kernel(s4_ref, bz_ref, pmmax_ref, out_ref):
    bz = bz_ref[0, 0]
    s4 = s4_ref[0]
    li = jax.lax.broadcasted_iota(jnp.int32, (_R, 128), 1)
    val = jnp.where(li < 32, s4[:, 0:1],
                    jnp.where(li < 64, s4[:, 1:2],
                              jnp.where(li < 96, s4[:, 2:3], s4[:, 3:4])))
    lz = li % 32
    mask = ((lz == bz) | (lz == bz + 1)) & (pmmax_ref[0, 0] > 0.0)
    out_ref[0] = jnp.where(mask, val, 0)


@jax.jit
def kernel(prob_map, scores, boxes, z_level, labels):
    del labels
    pmv = prob_map.reshape(_G, _R, 128)

    pad = _NP - _N
    sc_r = jnp.pad(scores, (0, pad), constant_values=-1.0).reshape(1, _NP)
    zl_r = jnp.pad(z_level, (0, pad), constant_values=-100).reshape(1, _NP)
    b_r = [jnp.pad(boxes[:, c], (0, pad)).reshape(1, _NP) for c in range(4)]
    sc_c = sc_r.reshape(_NP, 1)
    zl_c = zl_r.reshape(_NP, 1)
    b0_c = b_r[0].reshape(_NP, 1)
    b1_c = b_r[1].reshape(_NP, 1)

    pmbz4, boxarr, bz, pmmax, nv = pl.pallas_call(
        _prep_kernel,
        grid=(_G,),
        in_specs=[
            pl.BlockSpec((1, _R, 128), lambda s: (s, 0, 0)),
        ] + [pl.BlockSpec((1, _NP), lambda s: (0, 0))] * 6
        + [pl.BlockSpec((_NP, 1), lambda s: (0, 0))] * 4,
        out_specs=[
            pl.BlockSpec((1, _R, 4), lambda s: (s, 0, 0)),
            pl.BlockSpec((_NP, 16), lambda s: (0, 0)),
            pl.BlockSpec(memory_space=pltpu.SMEM),
            pl.BlockSpec(memory_space=pltpu.SMEM),
            pl.BlockSpec(memory_space=pltpu.SMEM),
        ],
        out_shape=[
            jax.ShapeDtypeStruct((_G, _R, 4), jnp.float32),
            jax.ShapeDtypeStruct((_NP, 16), jnp.int32),
            jax.ShapeDtypeStruct((1, 1), jnp.int32),
            jax.ShapeDtypeStruct((1, 1), jnp.float32),
            jax.ShapeDtypeStruct((1, 1), jnp.int32),
        ],
        scratch_shapes=[
            pltpu.SMEM((1,), jnp.int32),
            pltpu.SMEM((1,), jnp.float32),
        ],
    )(pmv, sc_r, *b_r, zl_r, sc_c, b0_c, b1_c, zl_c)

    pmbz = pmbz4.reshape(_X, _Y)

    sc_boxes = functools.partial(
        pl.kernel,
        mesh=plsc.VectorSubcoreMesh(core_axis_name="c", subcore_axis_name="s"),
        out_type=pltpu.HBM((_X, _Y), jnp.int32),
        scratch_types=[
            pltpu.VMEM((_NP // 8, 128), jnp.int32),
            pltpu.VMEM((64, 256), jnp.float32),
            pltpu.VMEM((16, _Y), jnp.int32),
        ],
    )(_sc_boxes_kernel)
    s_map = sc_boxes(
        pltpu.with_memory_space_constraint(pmbz, pltpu.HBM),
        pltpu.with_memory_space_constraint(
            boxarr.reshape(_NP // 8, 128), pltpu.HBM))

    s4 = s_map.reshape(_G, _R, 4)

    vol = pl.pallas_call(
        _assemble_kernel,
        grid=(_G,),
        in_specs=[
            pl.BlockSpec((1, _R, 4), lambda s: (s, 0, 0)),
            pl.BlockSpec(memory_space=pltpu.SMEM),
            pl.BlockSpec(memory_space=pltpu.SMEM),
        ],
        out_specs=pl.BlockSpec((1, _R, 128), lambda s: (s, 0, 0)),
        out_shape=jax.ShapeDtypeStruct((_G, _R, 128), jnp.int32),
    )(s4.astype(jnp.int32), bz, pmmax)

    return vol.reshape(1, 1, _X, _Y, _Z)


# speedup vs baseline: 1.3487x; 1.3487x over previous
import functools

import jax
import jax.numpy as jnp
from jax.experimental import pallas as pl
from jax.experimental.pallas import tpu as pltpu
from jax.experimental.pallas import tpu_sc as plsc

_X = 512
_Y = 512
_Z = 32
_N = 1000
_NP = 1024
_THR = 0.5
_G = 32
_R = (_X * _Y * _Z) // (128 * _G)
_P = (_X * _Y) // _G


def _prep_kernel(pm_ref, sc_r_ref, b0_r_ref, b1_r_ref, b2_r_ref, b3_r_ref,
                 zl_r_ref, sc_c_ref, b0_c_ref, b1_c_ref, zl_c_ref,
                 pmbz_ref, box_ref, bz_ref, pmmax_ref, nv_ref,
                 bz_s, pmmax_s):
    s = pl.program_id(0)

    @pl.when(s == 0)
    def _():
        sc_r = sc_r_ref[...]
        zl_r = zl_r_ref[...]
        sc_c = sc_c_ref[...]
        zl_c = zl_c_ref[...]

        keep_r = sc_r > _THR
        keep_c = sc_c > _THR
        kf_r = keep_r.astype(jnp.float32)
        kf_c = keep_c.astype(jnp.float32)
        K = jnp.sum(kf_r).astype(jnp.int32)

        best = jnp.float32(-1.0)
        bz = jnp.int32(0)
        for zz in range(_Z):
            m = keep_r & (zl_r == zz)
            cnt = jnp.sum(m.astype(jnp.float32))
            tot = jnp.sum(jnp.where(m, sc_r, 0.0))
            mean = jnp.where(cnt > 0, tot / jnp.maximum(cnt, 1.0),
                             jnp.float32(0.0))
            upd = mean > best
            bz = jnp.where(upd, jnp.int32(zz), bz)
            best = jnp.maximum(mean, best)
        bz_s[0] = bz
        bz_ref[0, 0] = bz

        i0i = jax.lax.broadcasted_iota(jnp.int32, (_NP, _NP), 0)
        i1i = jax.lax.broadcasted_iota(jnp.int32, (_NP, _NP), 1)
        i0 = i0i.astype(jnp.float32)
        i1 = i1i.astype(jnp.float32)

        le = (i0 <= i1).astype(jnp.float32)
        ge = (i0 >= i1).astype(jnp.float32)
        rank_r = jax.lax.dot_general(kf_r, le, (((1,), (0,)), ((), ())),
                                     preferred_element_type=jnp.float32)
        rank_c = jax.lax.dot_general(ge, kf_c, (((1,), (0,)), ((), ())),
                                     preferred_element_type=jnp.float32)

        sel_a = (keep_r & (rank_r - 1.0 == i0)).astype(jnp.float32)
        xk_c = jnp.sum(sel_a * b0_r_ref[...], axis=1, keepdims=True)
        ki_c = jnp.sum(sel_a * i1, axis=1, keepdims=True)
        sel_b = (keep_c & (rank_c - 1.0 == i1)).astype(jnp.float32)
        xk_r = jnp.sum(sel_b * b0_c_ref[...], axis=0, keepdims=True)
        ki_r = jnp.sum(sel_b * i0, axis=0, keepdims=True)

        jr = jax.lax.broadcasted_iota(jnp.int32, (1, _NP), 1)
        jc = jax.lax.broadcasted_iota(jnp.int32, (_NP, 1), 0)

        def passed(jv, b0, b1, sc, zl):
            return ((jv < K) & (b1 <= _X) & (b0 <= _Y) & (sc >= _THR)
                    & (zl >= bz - 2) & (zl <= bz + 2))

        p_r = passed(jr, b0_r_ref[...], b1_r_ref[...], sc_r, zl_r)
        p_c = passed(jc, b0_c_ref[...], b1_c_ref[...], sc_c, zl_c)

        lt = ((xk_c < xk_r) | ((xk_c == xk_r) & (ki_c < ki_r))
              ).astype(jnp.float32)
        cid = 2.0 + jax.lax.dot_general(
            p_c.astype(jnp.float32), lt, (((0,), (0,)), ((), ())),
            preferred_element_type=jnp.float32)
        cid = cid.astype(jnp.int32)

        y1 = b0_r_ref[...]
        x1 = b1_r_ref[...]
        y2 = b2_r_ref[...]
        x2 = b3_r_ref[...]
        fX = jnp.float32(_X)
        fY = jnp.float32(_Y)
        x2c = jnp.where(x2 > fX, fX - 1.0, x2)
        y2c = jnp.where((x2 <= fX) & (y2 > fY), fY - 1.0, y2)
        dx0 = jnp.where(x1 + 5.0 < 0.0, x1, jnp.float32(5.0))
        dy0 = jnp.where(y1 + 5.0 < 0.0, y1, jnp.float32(5.0))
        dx1 = jnp.where(x2c - 5.0 > fX, fX - x2c, jnp.float32(-5.0))
        dy1 = jnp.where(y2c - 5.0 > fY, fY - y2c, jnp.float32(-5.0))
        xa = jnp.round(x1 + dx0).astype(jnp.int32)
        xb = jnp.round(x2c + dx1).astype(jnp.int32)
        ya = jnp.round(y1 + dy0).astype(jnp.int32)
        yb = jnp.round(y2c + dy1).astype(jnp.int32)
        xa = jnp.maximum(xa, 0)
        ya = jnp.maximum(ya, 0)
        xb = jnp.minimum(xb, _X)
        yb = jnp.minimum(yb, _Y)
        valid = p_r & (xb > xa) & (yb > ya)
        vf = valid.astype(jnp.float32)
        nv_ref[0, 0] = jnp.sum(vf).astype(jnp.int32)

        vrank = jax.lax.dot_general(vf, le, (((1,), (0,)), ((), ())),
                                    preferred_element_type=jnp.float32)
        csel = (valid & (vrank - 1.0 == i0)).astype(jnp.float32)

        def compact(arr_row):
            return jnp.sum(csel * arr_row.astype(jnp.float32), axis=1,
                           keepdims=True).astype(jnp.int32)

        box_ref[:, 0:1] = compact(xa)
        box_ref[:, 1:2] = compact(xb)
        box_ref[:, 2:3] = compact(ya)
        box_ref[:, 3:4] = compact(yb)
        box_ref[:, 4:5] = compact(cid)
        box_ref[:, 5:16] = jnp.zeros((_NP, 11), jnp.int32)
        box_ref[0:1, 5:6] = nv_ref[0, 0].reshape(1, 1)

    blk = pm_ref[0, 0]
    blk_max = jnp.max(blk)

    @pl.when(s == 0)
    def _():
        pmmax_s[0] = blk_max

    @pl.when(s > 0)
    def _():
        pmmax_s[0] = jnp.maximum(pmmax_s[0], blk_max)

    @pl.when(s == _G - 1)
    def _():
        pmmax_ref[0, 0] = pmmax_s[0]

    lz = jax.lax.broadcasted_iota(jnp.int32, (16, _Y, _Z), 2)
    pmbz_ref[...] = jnp.sum(jnp.where(lz == bz_s[0], blk, 0.0), axis=2)


def _boxes_kernel(pm_ref, box_ref, nv_ref, s_out_ref):
    s_out_ref[...] = jnp.zeros((_X, _Y), jnp.int32)

    def body(i, carry):
        xa = box_ref[i, 0]
        xb = box_ref[i, 1]
        ya = box_ref[i, 2]
        yb = box_ref[i, 3]
        cid = box_ref[i, 4]
        bx = jnp.minimum((xa // 8) * 8, _X - 64)
        by = jnp.minimum((ya // 128) * 128, _Y - 256)
        win = pm_ref[pl.ds(bx, 64), pl.ds(by, 256)]
        scur = s_out_ref[pl.ds(bx, 64), pl.ds(by, 256)]
        ix = jax.lax.broadcasted_iota(jnp.int32, (64, 256), 0) + bx
        iy = jax.lax.broadcasted_iota(jnp.int32, (64, 256), 1) + by
        rect = (ix >= xa) & (ix < xb) & (iy >= ya) & (iy < yb)
        m = jnp.max(jnp.where(rect, win, -jnp.inf))
        eq = rect & (win == m)
        s_out_ref[pl.ds(bx, 64), pl.ds(by, 256)] = jnp.where(
            eq, jnp.maximum(scur, cid), scur)
        return carry

    jax.lax.fori_loop(0, nv_ref[0, 0], body, 0)


def _sc_boxes_kernel(pm_hbm, box_hbm, out_hbm, boxv, winv, sref):
    w = jax.lax.axis_index("s") * 2 + jax.lax.axis_index("c")
    x0 = w * 16
    pltpu.sync_copy(box_hbm, boxv)
    nv = boxv[0, pl.ds(0, 16)][5]
    il = jax.lax.iota(jnp.int32, 16)

    def zb(i, c):
        sref[i // 32, pl.ds((i % 32) * 16, 16)] = jnp.zeros((16,), jnp.int32)
        return c

    jax.lax.fori_loop(0, 512, zb, 0)

    def body(b, c):
        row = boxv[b // 8, pl.ds((b % 8) * 16, 16)]
        xa = row[0]
        xb = row[1]
        ya = row[2]
        yb = row[3]
        cid = row[4]
        xs = jnp.maximum(xa, x0)
        xe = jnp.minimum(xb, x0 + 16)

        @pl.when(xs < xe)
        def _():
            bx = jnp.minimum((xa // 8) * 8, _X - 64)
            by = jnp.minimum((ya // 128) * 128, _Y - 256)
            c0 = (ya - by) // 16
            pltpu.sync_copy(pm_hbm.at[pl.ds(bx, 64), pl.ds(by, 256)], winv)

            def rmax(r, acc):
                def cmax(ci, acc2):
                    y0 = by + (c0 + ci) * 16
                    v = winv[r, pl.ds((c0 + ci) * 16, 16)]
                    ym = (il + y0 >= ya) & (il + y0 < yb)
                    return jnp.maximum(acc2, jnp.where(ym, v, -jnp.inf))

                return jax.lax.fori_loop(0, 5, cmax, acc)

            acc = jax.lax.fori_loop(xa - bx, xb - bx, rmax,
                                    jnp.full((16,), -jnp.inf, jnp.float32))
            dnums = jax.lax.GatherDimensionNumbers(
                offset_dims=(), collapsed_slice_dims=(0,),
                start_index_map=(0,))
            for sh in (8, 4, 2, 1):
                peer = jax.lax.gather(
                    acc, (il ^ sh)[:, None], dnums, (1,),
                    mode=jax.lax.GatherScatterMode.PROMISE_IN_BOUNDS)
                acc = jnp.maximum(acc, peer)
            m = acc

            def rupd(x, c2):
                def cupd(ci, c3):
                    y0 = by + (c0 + ci) * 16
                    v = winv[x - bx, pl.ds((c0 + ci) * 16, 16)]
                    ym = (il + y0 >= ya) & (il + y0 < yb)
                    eq = ym & (v == m)
                    scur = sref[x - x0, pl.ds(y0, 16)]
                    sref[x - x0, pl.ds(y0, 16)] = jnp.where(
                        eq, jnp.maximum(scur, cid), scur)
                    return c3

                return jax.lax.fori_loop(0, 5, cupd, 0)

            jax.lax.fori_loop(xs, xe, rupd, 0)

        return c

    jax.lax.fori_loop(0, nv, body, 0)
    pltpu.sync_copy(sref, out_hbm.at[pl.ds(x0, 16)])


def _assemble_kernel(s_ref, bz_ref, pmmax_ref, out_ref):
    bz = bz_ref[0, 0]
    chunk_t = jnp.transpose(s_ref[...])
    lz = jax.lax.broadcasted_iota(jnp.int32, (_Y, _Z), 1)
    mask = ((lz == bz) | (lz == bz + 1)) & (pmmax_ref[0, 0] > 0.0)
    for xi in range(16):
        out_ref[0, 0, xi] = jnp.where(mask, chunk_t[:, xi:xi + 1], 0)


@jax.jit
def kernel(prob_map, scores, boxes, z_level, labels):
    del labels
    pad = _NP - _N
    sc_r = jnp.pad(scores, (0, pad), constant_values=-1.0).reshape(1, _NP)
    zl_r = jnp.pad(z_level, (0, pad), constant_values=-100).reshape(1, _NP)
    b_r = [jnp.pad(boxes[:, c], (0, pad)).reshape(1, _NP) for c in range(4)]
    sc_c = sc_r.reshape(_NP, 1)
    zl_c = zl_r.reshape(_NP, 1)
    b0_c = b_r[0].reshape(_NP, 1)
    b1_c = b_r[1].reshape(_NP, 1)

    pmbz, boxarr, bz, pmmax, nv = pl.pallas_call(
        _prep_kernel,
        grid=(_G,),
        in_specs=[
            pl.BlockSpec((1, 1, 16, _Y, _Z), lambda s: (0, 0, s, 0, 0)),
        ] + [pl.BlockSpec((1, _NP), lambda s: (0, 0))] * 6
        + [pl.BlockSpec((_NP, 1), lambda s: (0, 0))] * 4,
        out_specs=[
            pl.BlockSpec((16, _Y), lambda s: (s, 0)),
            pl.BlockSpec((_NP, 16), lambda s: (0, 0)),
            pl.BlockSpec(memory_space=pltpu.SMEM),
            pl.BlockSpec(memory_space=pltpu.SMEM),
            pl.BlockSpec(memory_space=pltpu.SMEM),
        ],
        out_shape=[
            jax.ShapeDtypeStruct((_X, _Y), jnp.float32),
            jax.ShapeDtypeStruct((_NP, 16), jnp.int32),
            jax.ShapeDtypeStruct((1, 1), jnp.int32),
            jax.ShapeDtypeStruct((1, 1), jnp.float32),
            jax.ShapeDtypeStruct((1, 1), jnp.int32),
        ],
        scratch_shapes=[
            pltpu.SMEM((1,), jnp.int32),
            pltpu.SMEM((1,), jnp.float32),
        ],
    )(prob_map, sc_r, *b_r, zl_r, sc_c, b0_c, b1_c, zl_c)

    sc_boxes = functools.partial(
        pl.kernel,
        mesh=plsc.VectorSubcoreMesh(core_axis_name="c", subcore_axis_name="s"),
        out_type=pltpu.HBM((_X, _Y), jnp.int32),
        scratch_types=[
            pltpu.VMEM((_NP // 8, 128), jnp.int32),
            pltpu.VMEM((64, 256), jnp.float32),
            pltpu.VMEM((16, _Y), jnp.int32),
        ],
    )(_sc_boxes_kernel)
    s_map = sc_boxes(
        pltpu.with_memory_space_constraint(pmbz, pltpu.HBM),
        pltpu.with_memory_space_constraint(
            boxarr.reshape(_NP // 8, 128), pltpu.HBM))

    vol = pl.pallas_call(
        _assemble_kernel,
        grid=(_G,),
        in_specs=[
            pl.BlockSpec((16, _Y), lambda s: (s, 0)),
            pl.BlockSpec(memory_space=pltpu.SMEM),
            pl.BlockSpec(memory_space=pltpu.SMEM),
        ],
        out_specs=pl.BlockSpec((1, 1, 16, _Y, _Z), lambda s: (0, 0, s, 0, 0)),
        out_shape=jax.ShapeDtypeStruct((1, 1, _X, _Y, _Z), jnp.int32),
    )(s_map, bz, pmmax)

    return vol


# speedup vs baseline: 1.3512x; 1.0018x over previous
import functools

import jax
import jax.numpy as jnp
from jax.experimental import pallas as pl
from jax.experimental.pallas import tpu as pltpu
from jax.experimental.pallas import tpu_sc as plsc

_X = 512
_Y = 512
_Z = 32
_N = 1000
_NP = 1024
_THR = 0.5
_G = 32
_R = (_X * _Y * _Z) // (128 * _G)
_P = (_X * _Y) // _G


def _prep_kernel(pm_ref, sc_r_ref, b0_r_ref, b1_r_ref, b2_r_ref, b3_r_ref,
                 zl_r_ref, sc_c_ref, b0_c_ref, b1_c_ref, zl_c_ref,
                 pmbz_ref, box_ref, bz_ref, pmmax_ref, nv_ref,
                 bz_s, pmmax_s):
    s = pl.program_id(0)

    @pl.when(s == 0)
    def _():
        sc_r = sc_r_ref[...]
        zl_r = zl_r_ref[...]
        sc_c = sc_c_ref[...]
        zl_c = zl_c_ref[...]

        keep_r = sc_r > _THR
        keep_c = sc_c > _THR
        kf_r = keep_r.astype(jnp.float32)
        kf_c = keep_c.astype(jnp.float32)
        K = jnp.sum(kf_r).astype(jnp.int32)

        best = jnp.float32(-1.0)
        bz = jnp.int32(0)
        for zz in range(_Z):
            m = keep_r & (zl_r == zz)
            cnt = jnp.sum(m.astype(jnp.float32))
            tot = jnp.sum(jnp.where(m, sc_r, 0.0))
            mean = jnp.where(cnt > 0, tot / jnp.maximum(cnt, 1.0),
                             jnp.float32(0.0))
            upd = mean > best
            bz = jnp.where(upd, jnp.int32(zz), bz)
            best = jnp.maximum(mean, best)
        bz_s[0] = bz
        bz_ref[0, 0] = bz

        i0i = jax.lax.broadcasted_iota(jnp.int32, (_NP, _NP), 0)
        i1i = jax.lax.broadcasted_iota(jnp.int32, (_NP, _NP), 1)
        i0 = i0i.astype(jnp.float32)
        i1 = i1i.astype(jnp.float32)

        le = (i0 <= i1).astype(jnp.float32)
        ge = (i0 >= i1).astype(jnp.float32)
        rank_r = jax.lax.dot_general(kf_r, le, (((1,), (0,)), ((), ())),
                                     preferred_element_type=jnp.float32)
        rank_c = jax.lax.dot_general(ge, kf_c, (((1,), (0,)), ((), ())),
                                     preferred_element_type=jnp.float32)

        sel_a = (keep_r & (rank_r - 1.0 == i0)).astype(jnp.float32)
        xk_c = jnp.sum(sel_a * b0_r_ref[...], axis=1, keepdims=True)
        ki_c = jnp.sum(sel_a * i1, axis=1, keepdims=True)
        sel_b = (keep_c & (rank_c - 1.0 == i1)).astype(jnp.float32)
        xk_r = jnp.sum(sel_b * b0_c_ref[...], axis=0, keepdims=True)
        ki_r = jnp.sum(sel_b * i0, axis=0, keepdims=True)

        jr = jax.lax.broadcasted_iota(jnp.int32, (1, _NP), 1)
        jc = jax.lax.broadcasted_iota(jnp.int32, (_NP, 1), 0)

        def passed(jv, b0, b1, sc, zl):
            return ((jv < K) & (b1 <= _X) & (b0 <= _Y) & (sc >= _THR)
                    & (zl >= bz - 2) & (zl <= bz + 2))

        p_r = passed(jr, b0_r_ref[...], b1_r_ref[...], sc_r, zl_r)
        p_c = passed(jc, b0_c_ref[...], b1_c_ref[...], sc_c, zl_c)

        lt = ((xk_c < xk_r) | ((xk_c == xk_r) & (ki_c < ki_r))
              ).astype(jnp.float32)
        cid = 2.0 + jax.lax.dot_general(
            p_c.astype(jnp.float32), lt, (((0,), (0,)), ((), ())),
            preferred_element_type=jnp.float32)
        cid = cid.astype(jnp.int32)

        y1 = b0_r_ref[...]
        x1 = b1_r_ref[...]
        y2 = b2_r_ref[...]
        x2 = b3_r_ref[...]
        fX = jnp.float32(_X)
        fY = jnp.float32(_Y)
        x2c = jnp.where(x2 > fX, fX - 1.0, x2)
        y2c = jnp.where((x2 <= fX) & (y2 > fY), fY - 1.0, y2)
        dx0 = jnp.where(x1 + 5.0 < 0.0, x1, jnp.float32(5.0))
        dy0 = jnp.where(y1 + 5.0 < 0.0, y1, jnp.float32(5.0))
        dx1 = jnp.where(x2c - 5.0 > fX, fX - x2c, jnp.float32(-5.0))
        dy1 = jnp.where(y2c - 5.0 > fY, fY - y2c, jnp.float32(-5.0))
        xa = jnp.round(x1 + dx0).astype(jnp.int32)
        xb = jnp.round(x2c + dx1).astype(jnp.int32)
        ya = jnp.round(y1 + dy0).astype(jnp.int32)
        yb = jnp.round(y2c + dy1).astype(jnp.int32)
        xa = jnp.maximum(xa, 0)
        ya = jnp.maximum(ya, 0)
        xb = jnp.minimum(xb, _X)
        yb = jnp.minimum(yb, _Y)
        valid = p_r & (xb > xa) & (yb > ya)
        vf = valid.astype(jnp.float32)
        nv_ref[0, 0] = jnp.sum(vf).astype(jnp.int32)

        vrank = jax.lax.dot_general(vf, le, (((1,), (0,)), ((), ())),
                                    preferred_element_type=jnp.float32)
        csel = (valid & (vrank - 1.0 == i0)).astype(jnp.float32)

        def compact(arr_row):
            return jnp.sum(csel * arr_row.astype(jnp.float32), axis=1,
                           keepdims=True).astype(jnp.int32)

        box_ref[:, 0:1] = compact(xa)
        box_ref[:, 1:2] = compact(xb)
        box_ref[:, 2:3] = compact(ya)
        box_ref[:, 3:4] = compact(yb)
        box_ref[:, 4:5] = compact(cid)
        box_ref[:, 5:16] = jnp.zeros((_NP, 11), jnp.int32)
        box_ref[0:1, 5:6] = nv_ref[0, 0].reshape(1, 1)

    blk = pm_ref[0, 0]
    blk_max = jnp.max(blk)

    @pl.when(s == 0)
    def _():
        pmmax_s[0] = blk_max

    @pl.when(s > 0)
    def _():
        pmmax_s[0] = jnp.maximum(pmmax_s[0], blk_max)

    @pl.when(s == _G - 1)
    def _():
        pmmax_ref[0, 0] = pmmax_s[0]

    lz = jax.lax.broadcasted_iota(jnp.int32, (16, _Y, _Z), 2)
    pmbz_ref[...] = jnp.sum(jnp.where(lz == bz_s[0], blk, 0.0), axis=2)


def _sc_boxes_kernel(pm_hbm, box_hbm, out_hbm, boxv, winv, sref):
    w = jax.lax.axis_index("s") * 2 + jax.lax.axis_index("c")
    x0 = w * 16
    pltpu.sync_copy(box_hbm, boxv)
    nv = boxv[0, pl.ds(0, 16)][5]
    il = jax.lax.iota(jnp.int32, 16)

    def zb(i, c):
        sref[i // 32, pl.ds((i % 32) * 16, 16)] = jnp.zeros((16,), jnp.int32)
        return c

    jax.lax.fori_loop(0, 512, zb, 0)

    def body(b, c):
        row = boxv[b // 8, pl.ds((b % 8) * 16, 16)]
        xa = row[0]
        xb = row[1]
        ya = row[2]
        yb = row[3]
        cid = row[4]
        xs = jnp.maximum(xa, x0)
        xe = jnp.minimum(xb, x0 + 16)

        @pl.when(xs < xe)
        def _():
            bx = jnp.minimum((xa // 8) * 8, _X - 64)
            by = jnp.minimum((ya // 128) * 128, _Y - 256)
            c0 = (ya - by) // 16
            pltpu.sync_copy(pm_hbm.at[pl.ds(bx, 64), pl.ds(by, 256)], winv)

            def rmax(r, acc):
                def cmax(ci, acc2):
                    y0 = by + (c0 + ci) * 16
                    v = winv[r, pl.ds((c0 + ci) * 16, 16)]
                    ym = (il + y0 >= ya) & (il + y0 < yb)
                    return jnp.maximum(acc2, jnp.where(ym, v, -jnp.inf))

                return jax.lax.fori_loop(0, 5, cmax, acc)

            acc = jax.lax.fori_loop(xa - bx, xb - bx, rmax,
                                    jnp.full((16,), -jnp.inf, jnp.float32))
            dnums = jax.lax.GatherDimensionNumbers(
                offset_dims=(), collapsed_slice_dims=(0,),
                start_index_map=(0,))
            for sh in (8, 4, 2, 1):
                peer = jax.lax.gather(
                    acc, (il ^ sh)[:, None], dnums, (1,),
                    mode=jax.lax.GatherScatterMode.PROMISE_IN_BOUNDS)
                acc = jnp.maximum(acc, peer)
            m = acc

            def rupd(x, c2):
                def cupd(ci, c3):
                    y0 = by + (c0 + ci) * 16
                    v = winv[x - bx, pl.ds((c0 + ci) * 16, 16)]
                    ym = (il + y0 >= ya) & (il + y0 < yb)
                    eq = ym & (v == m)
                    scur = sref[x - x0, pl.ds(y0, 16)]
                    sref[x - x0, pl.ds(y0, 16)] = jnp.where(
                        eq, jnp.maximum(scur, cid), scur)
                    return c3

                return jax.lax.fori_loop(0, 5, cupd, 0)

            jax.lax.fori_loop(xs, xe, rupd, 0)

        return c

    jax.lax.fori_loop(0, nv, body, 0)
    pltpu.sync_copy(sref, out_hbm.at[pl.ds(x0, 16)])


def _assemble_kernel(s_ref, bz_ref, pmmax_ref, out_ref):
    bz = bz_ref[0, 0]
    chunk_t = jnp.transpose(s_ref[...])
    lz = jax.lax.broadcasted_iota(jnp.int32, (_Y, _Z), 1)
    mask = ((lz == bz) | (lz == bz + 1)) & (pmmax_ref[0, 0] > 0.0)
    for xi in range(16):
        out_ref[0, 0, xi] = jnp.where(mask, chunk_t[:, xi:xi + 1], 0)


@jax.jit
def kernel(prob_map, scores, boxes, z_level, labels):
    del labels
    pad = _NP - _N
    sc_r = jnp.pad(scores, (0, pad), constant_values=-1.0).reshape(1, _NP)
    zl_r = jnp.pad(z_level, (0, pad), constant_values=-100).reshape(1, _NP)
    b_r = [jnp.pad(boxes[:, c], (0, pad)).reshape(1, _NP) for c in range(4)]
    sc_c = sc_r.reshape(_NP, 1)
    zl_c = zl_r.reshape(_NP, 1)
    b0_c = b_r[0].reshape(_NP, 1)
    b1_c = b_r[1].reshape(_NP, 1)

    pmbz, boxarr, bz, pmmax, nv = pl.pallas_call(
        _prep_kernel,
        grid=(_G,),
        in_specs=[
            pl.BlockSpec((1, 1, 16, _Y, _Z), lambda s: (0, 0, s, 0, 0)),
        ] + [pl.BlockSpec((1, _NP), lambda s: (0, 0))] * 6
        + [pl.BlockSpec((_NP, 1), lambda s: (0, 0))] * 4,
        out_specs=[
            pl.BlockSpec((16, _Y), lambda s: (s, 0)),
            pl.BlockSpec((_NP, 16), lambda s: (0, 0)),
            pl.BlockSpec(memory_space=pltpu.SMEM),
            pl.BlockSpec(memory_space=pltpu.SMEM),
            pl.BlockSpec(memory_space=pltpu.SMEM),
        ],
        out_shape=[
            jax.ShapeDtypeStruct((_X, _Y), jnp.float32),
            jax.ShapeDtypeStruct((_NP, 16), jnp.int32),
            jax.ShapeDtypeStruct((1, 1), jnp.int32),
            jax.ShapeDtypeStruct((1, 1), jnp.float32),
            jax.ShapeDtypeStruct((1, 1), jnp.int32),
        ],
        scratch_shapes=[
            pltpu.SMEM((1,), jnp.int32),
            pltpu.SMEM((1,), jnp.float32),
        ],
    )(prob_map, sc_r, *b_r, zl_r, sc_c, b0_c, b1_c, zl_c)

    sc_boxes = functools.partial(
        pl.kernel,
        mesh=plsc.VectorSubcoreMesh(core_axis_name="c", subcore_axis_name="s"),
        out_type=pltpu.HBM((_X, _Y), jnp.int32),
        scratch_types=[
            pltpu.VMEM((_NP // 8, 128), jnp.int32),
            pltpu.VMEM((64, 256), jnp.float32),
            pltpu.VMEM((16, _Y), jnp.int32),
        ],
    )(_sc_boxes_kernel)
    s_map = sc_boxes(
        pltpu.with_memory_space_constraint(pmbz, pltpu.HBM),
        pltpu.with_memory_space_constraint(
            boxarr.reshape(_NP // 8, 128), pltpu.HBM))

    vol = pl.pallas_call(
        _assemble_kernel,
        grid=(_G,),
        in_specs=[
            pl.BlockSpec((16, _Y), lambda s: (s, 0)),
            pl.BlockSpec(memory_space=pltpu.SMEM),
            pl.BlockSpec(memory_space=pltpu.SMEM),
        ],
        out_specs=pl.BlockSpec((1, 1, 16, _Y, _Z), lambda s: (0, 0, s, 0, 0)),
        out_shape=jax.ShapeDtypeStruct((1, 1, _X, _Y, _Z), jnp.int32),
    )(s_map, bz, pmmax)

    return vol
